# baseline (speedup 1.0000x reference)
import functools

import jax
import jax.numpy as jnp
from jax import lax
from jax.experimental import pallas as pl
from jax.experimental.pallas import tpu as pltpu
from jax.experimental.pallas import tpu_sc as plsc

N = 10000
E = 320000
F = 128
D2 = 48
DDEG = 16
NCLS = 40

NC, NS = 2, 16
NW = NC * NS
ZROWS = 632
ACC_ROWS = NS * ZROWS
OUT_TAIL = N - (NS - 1) * ZROWS

CFG_H = (96, 4, 108)
CFG_L2 = (128, 8, 80)
CFG_DEG = (128, 4, 80)

BM = 1000
GRID = N // BM


def _copy_out(c, s, acc_sh, out0, out1):
    r0 = s * ZROWS

    @pl.when(jnp.logical_and(c == 0, s < NS - 1))
    def _():
        pltpu.sync_copy(acc_sh.at[pl.ds(r0, ZROWS)],
                        out0.at[pl.ds(r0, ZROWS)])

    @pl.when(jnp.logical_and(c == 0, s == NS - 1))
    def _():
        pltpu.sync_copy(acc_sh.at[pl.ds(r0, OUT_TAIL)],
                        out0.at[pl.ds(r0, OUT_TAIL)])

    @pl.when(jnp.logical_and(c == 1, s < NS - 1))
    def _():
        pltpu.sync_copy(acc_sh.at[pl.ds(r0, ZROWS)],
                        out1.at[pl.ds(r0, ZROWS)])

    @pl.when(jnp.logical_and(c == 1, s == NS - 1))
    def _():
        pltpu.sync_copy(acc_sh.at[pl.ds(r0, OUT_TAIL)],
                        out1.at[pl.ds(r0, OUT_TAIL)])


@functools.lru_cache(maxsize=None)
def _make_sc_scatter(D, C, NBUF, CH):
    mesh = plsc.VectorSubcoreMesh(core_axis_name="c", subcore_axis_name="s",
                                  num_cores=NC, num_subcores=NS)

    @functools.partial(
        pl.kernel,
        out_type=(jax.ShapeDtypeStruct((N, D), jnp.float32),
                  jax.ShapeDtypeStruct((N, D), jnp.float32)),
        mesh=mesh,
        scratch_types=[
            [pltpu.VMEM((2, C), jnp.int32) for _ in range(NBUF)],
            [pltpu.VMEM((C, D), jnp.float32) for _ in range(NBUF)],
            pltpu.VMEM_SHARED((ACC_ROWS, D), jnp.float32),
            [pltpu.SemaphoreType.DMA for _ in range(NBUF)],
            [pltpu.SemaphoreType.DMA for _ in range(NBUF)],
        ],
        compiler_params=pltpu.CompilerParams(use_tc_tiling_on_sc=False),
    )
    def sc_scatter(hw_hbm, idx_hbm, zeros_hbm, out0, out1,
                   idx, rows, acc_sh, sem_i, sem_g):
        c = lax.axis_index("c")
        s = lax.axis_index("s")
        wid = s * NC + c

        pltpu.sync_copy(zeros_hbm, acc_sh.at[pl.ds(s * ZROWS, ZROWS)])
        plsc.subcore_barrier()

        gbase = wid * CH

        for k in range(NBUF):
            pltpu.async_copy(idx_hbm.at[gbase + k], idx[k], sem_i[k])
        for k in range(NBUF - 1):
            pltpu.make_async_copy(idx_hbm.at[gbase + k],
                                  idx[k], sem_i[k]).wait()
            pltpu.async_copy(hw_hbm.at[idx[k].at[0]], rows[k], sem_g[k])

        def stage(i, p):
            qg = (p + NBUF - 1) % NBUF

            pltpu.make_async_copy(hw_hbm.at[idx[p].at[0]],
                                  rows[p], sem_g[p]).wait()
            pltpu.sync_copy(rows[p], acc_sh.at[idx[p].at[1]], add=True)

            @pl.when(i + NBUF < CH)
            def _():
                pltpu.async_copy(idx_hbm.at[gbase + i + NBUF],
                                 idx[p], sem_i[p])

            @pl.when(i + NBUF - 1 < CH)
            def _():
                pltpu.make_async_copy(idx_hbm.at[gbase + i + NBUF - 1],
                                      idx[qg], sem_i[qg]).wait()
                pltpu.async_copy(hw_hbm.at[idx[qg].at[0]],
                                 rows[qg], sem_g[qg])

        def body(j, carry):
            for p in range(NBUF):
                stage(NBUF * j + p, p)
            return carry

        lax.fori_loop(0, CH // NBUF, body, 0)
        plsc.subcore_barrier()
        _copy_out(c, s, acc_sh, out0, out1)

    return sc_scatter


@functools.lru_cache(maxsize=None)
def _make_sc_deg():
    C, NBUF, CH = CFG_DEG
    mesh = plsc.VectorSubcoreMesh(core_axis_name="c", subcore_axis_name="s",
                                  num_cores=NC, num_subcores=NS)

    @functools.partial(
        pl.kernel,
        out_type=(jax.ShapeDtypeStruct((N, DDEG), jnp.float32),
                  jax.ShapeDtypeStruct((N, DDEG), jnp.float32)),
        mesh=mesh,
        scratch_types=[
            [pltpu.VMEM((2, C), jnp.int32) for _ in range(NBUF)],
            pltpu.VMEM((C, DDEG), jnp.float32),
            pltpu.VMEM_SHARED((ACC_ROWS, DDEG), jnp.float32),
            [pltpu.SemaphoreType.DMA for _ in range(NBUF)],
            pltpu.SemaphoreType.DMA,
        ],
        compiler_params=pltpu.CompilerParams(use_tc_tiling_on_sc=False),
    )
    def sc_deg(idx_hbm, ones_hbm, zeros_hbm, out0, out1,
               idx, ones_v, acc_sh, sem_i, sem_s):
        c = lax.axis_index("c")
        s = lax.axis_index("s")
        wid = s * NC + c

        pltpu.sync_copy(ones_hbm, ones_v)
        pltpu.sync_copy(zeros_hbm, acc_sh.at[pl.ds(s * ZROWS, ZROWS)])
        plsc.subcore_barrier()

        gbase = wid * CH
        for k in range(NBUF):
            pltpu.async_copy(idx_hbm.at[gbase + k], idx[k], sem_i[k])

        def stage(i, p):
            pltpu.make_async_copy(idx_hbm.at[gbase + i],
                                  idx[p], sem_i[p]).wait()
            pltpu.sync_copy(ones_v, acc_sh.at[idx[p].at[1]], add=True)

            @pl.when(i + NBUF < CH)
            def _():
                pltpu.async_copy(idx_hbm.at[gbase + i + NBUF],
                                 idx[p], sem_i[p])

        def body(j, carry):
            for p in range(NBUF):
                stage(NBUF * j + p, p)
            return carry

        lax.fori_loop(0, CH // NBUF, body, 0)
        plsc.subcore_barrier()
        _copy_out(c, s, acc_sh, out0, out1)

    return sc_deg


def _pack_idx(src, dst, C, CH):
    e_pad = NW * C * CH
    src_p = jnp.concatenate([src, jnp.zeros((e_pad - E,), jnp.int32)])
    dst_p = jnp.concatenate([dst, jnp.full((e_pad - E,), N, jnp.int32)])
    return jnp.stack([src_p.reshape(-1, C), dst_p.reshape(-1, C)], axis=1)


def _tc1_body(x_ref, w_ref, out_ref):
    out_ref[...] = jnp.dot(x_ref[...], w_ref[...],
                           preferred_element_type=jnp.float32)


def _tc1(x, w0):
    return pl.pallas_call(
        _tc1_body,
        grid=(GRID,),
        in_specs=[
            pl.BlockSpec((BM, F), lambda m: (m, 0)),
            pl.BlockSpec((F, F), lambda m: (0, 0)),
        ],
        out_specs=pl.BlockSpec((BM, F), lambda m: (m, 0)),
        out_shape=jax.ShapeDtypeStruct((N, F), jnp.float32),
    )(x, w0)


def _tc2_body(a0_ref, a1_ref, g0_ref, g1_ref, w_ref, b_ref, hw_ref, dinv_ref):
    deg = g0_ref[:, 0:1] + g1_ref[:, 0:1]
    dinv = 1.0 / jnp.maximum(deg, 1.0)
    h = jnp.maximum((a0_ref[...] + a1_ref[...]) * dinv + b_ref[...], 0.0)
    hw_ref[...] = jnp.dot(h, w_ref[...], preferred_element_type=jnp.float32)
    dinv_ref[...] = jnp.broadcast_to(dinv, (BM, F))


def _tc2(a0, a1, g0, g1, w1, b0):
    return pl.pallas_call(
        _tc2_body,
        grid=(GRID,),
        in_specs=[
            pl.BlockSpec((BM, F), lambda m: (m, 0)),
            pl.BlockSpec((BM, F), lambda m: (m, 0)),
            pl.BlockSpec((BM, DDEG), lambda m: (m, 0)),
            pl.BlockSpec((BM, DDEG), lambda m: (m, 0)),
            pl.BlockSpec((F, F), lambda m: (0, 0)),
            pl.BlockSpec((1, F), lambda m: (0, 0)),
        ],
        out_specs=[
            pl.BlockSpec((BM, F), lambda m: (m, 0)),
            pl.BlockSpec((BM, F), lambda m: (m, 0)),
        ],
        out_shape=[
            jax.ShapeDtypeStruct((N, F), jnp.float32),
            jax.ShapeDtypeStruct((N, F), jnp.float32),
        ],
    )(a0, a1, g0, g1, w1, b0)


def _tc3_body(a0_ref, a1_ref, dinv_ref, w_ref, b_ref, out_ref):
    h = jnp.maximum((a0_ref[...] + a1_ref[...]) * dinv_ref[...] + b_ref[...],
                    0.0)
    out_ref[...] = jnp.dot(h, w_ref[...], preferred_element_type=jnp.float32)


def _tc3(a0, a1, dinv, w2p, b1):
    return pl.pallas_call(
        _tc3_body,
        grid=(GRID,),
        in_specs=[
            pl.BlockSpec((BM, F), lambda m: (m, 0)),
            pl.BlockSpec((BM, F), lambda m: (m, 0)),
            pl.BlockSpec((BM, F), lambda m: (m, 0)),
            pl.BlockSpec((F, D2), lambda m: (0, 0)),
            pl.BlockSpec((1, F), lambda m: (0, 0)),
        ],
        out_specs=pl.BlockSpec((BM, D2), lambda m: (m, 0)),
        out_shape=jax.ShapeDtypeStruct((N, D2), jnp.float32),
    )(a0, a1, dinv, w2p, b1)


def _tc4_body(a0_ref, a1_ref, dinv_ref, b_ref, out_ref):
    out_ref[...] = ((a0_ref[...] + a1_ref[...]) * dinv_ref[:, :D2]
                    + b_ref[...])


def _tc4(a0, a1, dinv, b2p):
    return pl.pallas_call(
        _tc4_body,
        grid=(GRID,),
        in_specs=[
            pl.BlockSpec((BM, D2), lambda m: (m, 0)),
            pl.BlockSpec((BM, D2), lambda m: (m, 0)),
            pl.BlockSpec((BM, F), lambda m: (m, 0)),
            pl.BlockSpec((1, D2), lambda m: (0, 0)),
        ],
        out_specs=pl.BlockSpec((BM, D2), lambda m: (m, 0)),
        out_shape=jax.ShapeDtypeStruct((N, D2), jnp.float32),
    )(a0, a1, dinv, b2p)


def kernel(features, edge_index, W0, b0, W1, b1, W2, b2):
    src = edge_index[0]
    dst = edge_index[1]
    idx_h = _pack_idx(src, dst, CFG_H[0], CFG_H[2])
    idx_l2 = _pack_idx(src, dst, CFG_L2[0], CFG_L2[2])

    w2p = jnp.pad(W2, ((0, 0), (0, D2 - NCLS)))
    b2p = jnp.pad(b2, (0, D2 - NCLS))
    zeros_f = jnp.zeros((ZROWS, F), jnp.float32)

    ga, gb = _make_sc_deg()(idx_l2, jnp.ones((CFG_DEG[0], DDEG), jnp.float32),
                            jnp.zeros((ZROWS, DDEG), jnp.float32))

    hw0 = _tc1(features, W0)
    p0a, p0b = _make_sc_scatter(F, *CFG_H)(hw0, idx_h, zeros_f)
    hw1, dinv = _tc2(p0a, p0b, ga, gb, W1, b0[None, :])
    p1a, p1b = _make_sc_scatter(F, *CFG_H)(hw1, idx_h, zeros_f)
    hw2 = _tc3(p1a, p1b, dinv, w2p, b1[None, :])
    p2a, p2b = _make_sc_scatter(D2, *CFG_L2)(hw2, idx_l2,
                                             jnp.zeros((ZROWS, D2),
                                                       jnp.float32))
    out = _tc4(p2a, p2b, dinv, b2p[None, :])
    return out[:, :NCLS]


# speedup vs baseline: 2.7580x; 2.7580x over previous
import functools

import jax
import jax.numpy as jnp
from jax import lax
from jax.experimental import pallas as pl
from jax.experimental.pallas import tpu as pltpu
from jax.experimental.pallas import tpu_sc as plsc

N = 10000
E = 320000
F = 128
FH = 64
D2 = 64
D2H = 32
DDEG = 16
NCLS = 40

NC, NS = 2, 16
NW = NC * NS
C = 128
CH_T = 160
CH_DEG = 80
E_PAD = NS * CH_T * C
ZROWS = 632
ACC_ROWS = NS * ZROWS
ROW_TAIL = N - (NS - 1) * ZROWS

BM = 1000
GRID = N // BM


def _rowwise(fn):
    def run(s):
        @pl.when(s < NS - 1)
        def _():
            fn(s * ZROWS, ZROWS)

        @pl.when(s == NS - 1)
        def _():
            fn((NS - 1) * ZROWS, ROW_TAIL)
    return run


@functools.lru_cache(maxsize=None)
def _make_sc_agg(DH, NBUF):
    mesh = plsc.VectorSubcoreMesh(core_axis_name="c", subcore_axis_name="s",
                                  num_cores=NC, num_subcores=NS)

    @functools.partial(
        pl.kernel,
        out_type=(jax.ShapeDtypeStruct((N, DH), jnp.float32),
                  jax.ShapeDtypeStruct((N, DH), jnp.float32)),
        mesh=mesh,
        scratch_types=[
            [pltpu.VMEM((2, C), jnp.int32) for _ in range(NBUF)],
            [pltpu.VMEM((C, DH), jnp.float32) for _ in range(NBUF)],
            pltpu.VMEM_SHARED((N, DH), jnp.float32),
            pltpu.VMEM_SHARED((ACC_ROWS, DH), jnp.float32),
            [pltpu.SemaphoreType.DMA for _ in range(NBUF)],
            [pltpu.SemaphoreType.DMA for _ in range(NBUF)],
        ],
        compiler_params=pltpu.CompilerParams(use_tc_tiling_on_sc=False),
    )
    def sc_agg(lo_hbm, hi_hbm, idx_hbm, zeros_hbm, out0, out1,
               idx, rows, tab_sh, acc_sh, sem_i, sem_g):
        c = lax.axis_index("c")
        s = lax.axis_index("s")

        @pl.when(c == 0)
        def _():
            _rowwise(lambda r0, nr: pltpu.sync_copy(
                lo_hbm.at[pl.ds(r0, nr)], tab_sh.at[pl.ds(r0, nr)]))(s)

        @pl.when(c == 1)
        def _():
            _rowwise(lambda r0, nr: pltpu.sync_copy(
                hi_hbm.at[pl.ds(r0, nr)], tab_sh.at[pl.ds(r0, nr)]))(s)

        pltpu.sync_copy(zeros_hbm, acc_sh.at[pl.ds(s * ZROWS, ZROWS)])
        plsc.subcore_barrier()

        gbase = s * CH_T

        for k in range(NBUF):
            pltpu.async_copy(idx_hbm.at[gbase + k], idx[k], sem_i[k])
        for k in range(NBUF - 1):
            pltpu.make_async_copy(idx_hbm.at[gbase + k],
                                  idx[k], sem_i[k]).wait()
            pltpu.async_copy(tab_sh.at[idx[k].at[0]], rows[k], sem_g[k])

        def stage(i, p):
            qg = (p + NBUF - 1) % NBUF

            pltpu.make_async_copy(tab_sh.at[idx[p].at[0]],
                                  rows[p], sem_g[p]).wait()
            pltpu.sync_copy(rows[p], acc_sh.at[idx[p].at[1]], add=True)

            @pl.when(i + NBUF < CH_T)
            def _():
                pltpu.async_copy(idx_hbm.at[gbase + i + NBUF],
                                 idx[p], sem_i[p])

            @pl.when(i + NBUF - 1 < CH_T)
            def _():
                pltpu.make_async_copy(idx_hbm.at[gbase + i + NBUF - 1],
                                      idx[qg], sem_i[qg]).wait()
                pltpu.async_copy(tab_sh.at[idx[qg].at[0]],
                                 rows[qg], sem_g[qg])

        def body(j, carry):
            for p in range(NBUF):
                stage(NBUF * j + p, p)
            return carry

        lax.fori_loop(0, CH_T // NBUF, body, 0)
        plsc.subcore_barrier()

        @pl.when(c == 0)
        def _():
            _rowwise(lambda r0, nr: pltpu.sync_copy(
                acc_sh.at[pl.ds(r0, nr)], out0.at[pl.ds(r0, nr)]))(s)

        @pl.when(c == 1)
        def _():
            _rowwise(lambda r0, nr: pltpu.sync_copy(
                acc_sh.at[pl.ds(r0, nr)], out1.at[pl.ds(r0, nr)]))(s)

    return sc_agg


@functools.lru_cache(maxsize=None)
def _make_sc_deg():
    NBUF = 4
    mesh = plsc.VectorSubcoreMesh(core_axis_name="c", subcore_axis_name="s",
                                  num_cores=NC, num_subcores=NS)

    @functools.partial(
        pl.kernel,
        out_type=(jax.ShapeDtypeStruct((N, DDEG), jnp.float32),
                  jax.ShapeDtypeStruct((N, DDEG), jnp.float32)),
        mesh=mesh,
        scratch_types=[
            [pltpu.VMEM((2, C), jnp.int32) for _ in range(NBUF)],
            pltpu.VMEM((C, DDEG), jnp.float32),
            pltpu.VMEM_SHARED((ACC_ROWS, DDEG), jnp.float32),
            [pltpu.SemaphoreType.DMA for _ in range(NBUF)],
            pltpu.SemaphoreType.DMA,
        ],
        compiler_params=pltpu.CompilerParams(use_tc_tiling_on_sc=False),
    )
    def sc_deg(idx_hbm, ones_hbm, zeros_hbm, out0, out1,
               idx, ones_v, acc_sh, sem_i, sem_s):
        c = lax.axis_index("c")
        s = lax.axis_index("s")
        wid = s * NC + c

        pltpu.sync_copy(ones_hbm, ones_v)
        pltpu.sync_copy(zeros_hbm, acc_sh.at[pl.ds(s * ZROWS, ZROWS)])
        plsc.subcore_barrier()

        gbase = wid * CH_DEG
        for k in range(NBUF):
            pltpu.async_copy(idx_hbm.at[gbase + k], idx[k], sem_i[k])

        def stage(i, p):
            pltpu.make_async_copy(idx_hbm.at[gbase + i],
                                  idx[p], sem_i[p]).wait()
            pltpu.sync_copy(ones_v, acc_sh.at[idx[p].at[1]], add=True)

            @pl.when(i + NBUF < CH_DEG)
            def _():
                pltpu.async_copy(idx_hbm.at[gbase + i + NBUF],
                                 idx[p], sem_i[p])

        def body(j, carry):
            for p in range(NBUF):
                stage(NBUF * j + p, p)
            return carry

        lax.fori_loop(0, CH_DEG // NBUF, body, 0)
        plsc.subcore_barrier()

        @pl.when(c == 0)
        def _():
            _rowwise(lambda r0, nr: pltpu.sync_copy(
                acc_sh.at[pl.ds(r0, nr)], out0.at[pl.ds(r0, nr)]))(s)

        @pl.when(c == 1)
        def _():
            _rowwise(lambda r0, nr: pltpu.sync_copy(
                acc_sh.at[pl.ds(r0, nr)], out1.at[pl.ds(r0, nr)]))(s)

    return sc_deg


def _pack_idx(src, dst):
    src_p = jnp.concatenate([src, jnp.zeros((E_PAD - E,), jnp.int32)])
    dst_p = jnp.concatenate([dst, jnp.full((E_PAD - E,), N, jnp.int32)])
    return jnp.stack([src_p.reshape(-1, C), dst_p.reshape(-1, C)], axis=1)


def _tc1_body(x_ref, w_ref, lo_ref, hi_ref):
    xw = jnp.dot(x_ref[...], w_ref[...], preferred_element_type=jnp.float32)
    lo_ref[...] = xw[:, :FH]
    hi_ref[...] = xw[:, FH:]


def _tc1(x, w0):
    return pl.pallas_call(
        _tc1_body,
        grid=(GRID,),
        in_specs=[
            pl.BlockSpec((BM, F), lambda m: (m, 0)),
            pl.BlockSpec((F, F), lambda m: (0, 0)),
        ],
        out_specs=[
            pl.BlockSpec((BM, FH), lambda m: (m, 0)),
            pl.BlockSpec((BM, FH), lambda m: (m, 0)),
        ],
        out_shape=[
            jax.ShapeDtypeStruct((N, FH), jnp.float32),
            jax.ShapeDtypeStruct((N, FH), jnp.float32),
        ],
    )(x, w0)


def _tc2_body(lo_ref, hi_ref, g0_ref, g1_ref, w_ref, b_ref,
              lo_out, hi_out, dinv_ref):
    deg = g0_ref[:, 0:1] + g1_ref[:, 0:1]
    dinv = 1.0 / jnp.maximum(deg, 1.0)
    agg = jnp.concatenate([lo_ref[...], hi_ref[...]], axis=1)
    h = jnp.maximum(agg * dinv + b_ref[...], 0.0)
    hw = jnp.dot(h, w_ref[...], preferred_element_type=jnp.float32)
    lo_out[...] = hw[:, :FH]
    hi_out[...] = hw[:, FH:]
    dinv_ref[...] = jnp.broadcast_to(dinv, (BM, F))


def _tc2(lo, hi, g0, g1, w1, b0):
    return pl.pallas_call(
        _tc2_body,
        grid=(GRID,),
        in_specs=[
            pl.BlockSpec((BM, FH), lambda m: (m, 0)),
            pl.BlockSpec((BM, FH), lambda m: (m, 0)),
            pl.BlockSpec((BM, DDEG), lambda m: (m, 0)),
            pl.BlockSpec((BM, DDEG), lambda m: (m, 0)),
            pl.BlockSpec((F, F), lambda m: (0, 0)),
            pl.BlockSpec((1, F), lambda m: (0, 0)),
        ],
        out_specs=[
            pl.BlockSpec((BM, FH), lambda m: (m, 0)),
            pl.BlockSpec((BM, FH), lambda m: (m, 0)),
            pl.BlockSpec((BM, F), lambda m: (m, 0)),
        ],
        out_shape=[
            jax.ShapeDtypeStruct((N, FH), jnp.float32),
            jax.ShapeDtypeStruct((N, FH), jnp.float32),
            jax.ShapeDtypeStruct((N, F), jnp.float32),
        ],
    )(lo, hi, g0, g1, w1, b0)


def _tc3_body(lo_ref, hi_ref, dinv_ref, w_ref, b_ref, lo_out, hi_out):
    agg = jnp.concatenate([lo_ref[...], hi_ref[...]], axis=1)
    h = jnp.maximum(agg * dinv_ref[...] + b_ref[...], 0.0)
    hw = jnp.dot(h, w_ref[...], preferred_element_type=jnp.float32)
    lo_out[...] = hw[:, :D2H]
    hi_out[...] = hw[:, D2H:]


def _tc3(lo, hi, dinv, w2p, b1):
    return pl.pallas_call(
        _tc3_body,
        grid=(GRID,),
        in_specs=[
            pl.BlockSpec((BM, FH), lambda m: (m, 0)),
            pl.BlockSpec((BM, FH), lambda m: (m, 0)),
            pl.BlockSpec((BM, F), lambda m: (m, 0)),
            pl.BlockSpec((F, D2), lambda m: (0, 0)),
            pl.BlockSpec((1, F), lambda m: (0, 0)),
        ],
        out_specs=[
            pl.BlockSpec((BM, D2H), lambda m: (m, 0)),
            pl.BlockSpec((BM, D2H), lambda m: (m, 0)),
        ],
        out_shape=[
            jax.ShapeDtypeStruct((N, D2H), jnp.float32),
            jax.ShapeDtypeStruct((N, D2H), jnp.float32),
        ],
    )(lo, hi, dinv, w2p, b1)


def _tc4_body(lo_ref, hi_ref, dinv_ref, b_ref, out_ref):
    agg = jnp.concatenate([lo_ref[...], hi_ref[...]], axis=1)
    out_ref[...] = agg * dinv_ref[:, :D2] + b_ref[...]


def _tc4(lo, hi, dinv, b2p):
    return pl.pallas_call(
        _tc4_body,
        grid=(GRID,),
        in_specs=[
            pl.BlockSpec((BM, D2H), lambda m: (m, 0)),
            pl.BlockSpec((BM, D2H), lambda m: (m, 0)),
            pl.BlockSpec((BM, F), lambda m: (m, 0)),
            pl.BlockSpec((1, D2), lambda m: (0, 0)),
        ],
        out_specs=pl.BlockSpec((BM, D2), lambda m: (m, 0)),
        out_shape=jax.ShapeDtypeStruct((N, D2), jnp.float32),
    )(lo, hi, dinv, b2p)


def kernel(features, edge_index, W0, b0, W1, b1, W2, b2):
    src = edge_index[0]
    dst = edge_index[1]
    idx_p = _pack_idx(src, dst)

    w2p = jnp.pad(W2, ((0, 0), (0, D2 - NCLS)))
    b2p = jnp.pad(b2, (0, D2 - NCLS))
    zeros_h = jnp.zeros((ZROWS, FH), jnp.float32)

    ga, gb = _make_sc_deg()(idx_p, jnp.ones((C, DDEG), jnp.float32),
                            jnp.zeros((ZROWS, DDEG), jnp.float32))

    lo0, hi0 = _tc1(features, W0)
    a0l, a0h = _make_sc_agg(FH, 4)(lo0, hi0, idx_p, zeros_h)
    lo1, hi1, dinv = _tc2(a0l, a0h, ga, gb, W1, b0[None, :])
    a1l, a1h = _make_sc_agg(FH, 4)(lo1, hi1, idx_p, zeros_h)
    lo2, hi2 = _tc3(a1l, a1h, dinv, w2p, b1[None, :])
    a2l, a2h = _make_sc_agg(D2H, 4)(lo2, hi2, idx_p,
                                    jnp.zeros((ZROWS, D2H), jnp.float32))
    out = _tc4(a2l, a2h, dinv, b2p[None, :])
    return out[:, :NCLS]
